# baseline (speedup 1.0000x reference)
import functools

import jax
import jax.numpy as jnp
from jax import lax
from jax.experimental import pallas as pl
from jax.experimental.pallas import tpu as pltpu
from jax.experimental.pallas import tpu_sc as plsc

N = 320000
D = 128
S = 10000


BN = 16000
NB = N // BN


def _scores_body(x_ref, w_ref, s_ref, m_ref, z_ref):
    i = pl.program_id(0)
    xb = x_ref[...]
    wr = w_ref[...]
    sb = lax.dot_general(wr, xb, (((1,), (1,)), ((), ())),
                         preferred_element_type=jnp.float32)
    s_ref[0] = sb
    bm = jnp.max(sb)

    @pl.when(i == 0)
    def _():
        m_ref[0, 0] = bm
        z_ref[0, 0] = jnp.sum(jnp.exp(sb - bm))

    @pl.when(i > 0)
    def _():
        m_old = m_ref[0, 0]
        m_new = jnp.maximum(m_old, bm)
        z_ref[0, 0] = (z_ref[0, 0] * jnp.exp(m_old - m_new)
                       + jnp.sum(jnp.exp(sb - m_new)))
        m_ref[0, 0] = m_new


def _compute_scores(x, w):
    return pl.pallas_call(
        _scores_body,
        grid=(NB,),
        in_specs=[
            pl.BlockSpec((BN, D), lambda i: (i, 0)),
            pl.BlockSpec((1, D), lambda i: (0, 0)),
        ],
        out_specs=[
            pl.BlockSpec((1, 1, BN), lambda i: (i, 0, 0)),
            pl.BlockSpec((1, 1), lambda i: (0, 0), memory_space=pltpu.SMEM),
            pl.BlockSpec((1, 1), lambda i: (0, 0), memory_space=pltpu.SMEM),
        ],
        out_shape=[
            jax.ShapeDtypeStruct((NB, 1, BN), jnp.float32),
            jax.ShapeDtypeStruct((1, 1), jnp.float32),
            jax.ShapeDtypeStruct((1, 1), jnp.float32),
        ],
    )(x, w)


NC = 2
NS = 16
TPT = N // (NC * NS)
RB = 80
NBLK = TPT // RB
SPS = 624
SREM = S - NS * SPS


NRING = 3
CAP = 16


def _sc_body(x_hbm, s_hbm, idx_hbm, mz_hbm, zeros_hbm, out_hbm,
             xb, yb, sb, ib, ic, mzb, cnts, acc_sh,
             sin0, sin1, sin2, ssc0, ssc1, ssc2):
    c = lax.axis_index("c")
    sid = lax.axis_index("s")
    wid = c * NS + sid
    base = wid * TPT
    iota = lax.broadcasted_iota(jnp.int32, (16,), 0)

    pltpu.sync_copy(zeros_hbm.at[pl.ds(sid * SPS, SPS)],
                    acc_sh.at[pl.ds(sid * SPS, SPS)])

    @pl.when(sid == 0)
    def _():
        pltpu.sync_copy(zeros_hbm.at[pl.ds(NS * SPS, SREM)],
                        acc_sh.at[pl.ds(NS * SPS, SREM)])

    pltpu.sync_copy(mz_hbm, mzb)
    plsc.subcore_barrier()

    mv = mzb[...]
    m = mv[0]
    z = mv[1]
    sems_in = (sin0, sin1, sin2)
    sems_sc = (ssc0, ssc1, ssc2)

    def start_in(g, k):
        b0 = base + g * RB
        pltpu.async_copy(x_hbm.at[pl.ds(b0, RB)], xb.at[k], sems_in[k])
        pltpu.async_copy(s_hbm.at[pl.ds(b0, RB)], sb.at[k], sems_in[k])
        pltpu.async_copy(idx_hbm.at[pl.ds(b0, RB)], ib.at[k], sems_in[k])

    def wait_in(k):
        pltpu.make_async_copy(x_hbm.at[pl.ds(0, RB)], xb.at[k],
                              sems_in[k]).wait()
        pltpu.make_async_copy(s_hbm.at[pl.ds(0, RB)], sb.at[k],
                              sems_in[k]).wait()
        pltpu.make_async_copy(idx_hbm.at[pl.ds(0, RB)], ib.at[k],
                              sems_in[k]).wait()

    def compute(k):
        iv0 = ib[k, pl.ds(0, 16)]
        zero16 = jnp.zeros((16,), jnp.float32)
        init = (jnp.int32(0), iv0[0], jnp.full((16,), S, jnp.int32))
        init = init + (zero16,) * (D // 16)

        def grp(gg, carry):
            cnt, cur, idvec = carry[0], carry[1], carry[2]
            accs = list(carry[3:])
            sv = sb[k, pl.ds(gg * 16, 16)]
            wv = jnp.exp(sv - m) / z
            iv = ib[k, pl.ds(gg * 16, 16)]
            for r in range(16):
                row = gg * 16 + r
                idr = iv[r]
                wr = wv[r]
                ne = idr != cur
                cnt = cnt + ne.astype(jnp.int32)
                slot = jnp.minimum(cnt, CAP - 1)
                idvec = jnp.where(iota == slot, idr, idvec)
                for j in range(D // 16):
                    xw = xb[k, row, pl.ds(j * 16, 16)] * wr
                    a = jnp.where(ne, xw, accs[j] + xw)
                    yb[k, slot, pl.ds(j * 16, 16)] = a
                    accs[j] = a
                cur = idr
            return (cnt, cur, idvec) + tuple(accs)

        out = lax.fori_loop(0, RB // 16, grp, init)
        cnt, idvec = out[0], out[2]
        ic[k, pl.ds(0, 16)] = idvec
        cnts[k] = cnt
        return cnt

    def rescale_plain(k):
        def grp(gg, carry):
            sv = sb[k, pl.ds(gg * 16, 16)]
            wv = jnp.exp(sv - m) / z
            for r in range(16):
                row = gg * 16 + r
                wr = wv[r]
                for j in range(D // 16):
                    xb[k, row, pl.ds(j * 16, 16)] = (
                        xb[k, row, pl.ds(j * 16, 16)] * wr)
            return carry

        lax.fori_loop(0, RB // 16, grp, 0)

    def start_sc(k, cnt):
        @pl.when(cnt < CAP)
        def _():
            pltpu.async_copy(yb.at[k], acc_sh.at[ic.at[k]], sems_sc[k],
                             add=True)

        @pl.when(cnt >= CAP)
        def _():
            rescale_plain(k)
            pltpu.async_copy(xb.at[k], acc_sh.at[ib.at[k]], sems_sc[k],
                             add=True)

    def wait_sc(k):
        cv = cnts[k]

        @pl.when(cv < CAP)
        def _():
            pltpu.make_async_copy(yb.at[k], acc_sh.at[ic.at[k]],
                                  sems_sc[k]).wait()

        @pl.when(cv >= CAP)
        def _():
            pltpu.make_async_copy(xb.at[k], acc_sh.at[ib.at[k]],
                                  sems_sc[k]).wait()

    start_in(0, 0)
    start_in(1, 1)

    def triple(p, carry):
        for k in range(NRING):
            g = 3 * p + k
            wait_in(k)
            cnt = compute(k)
            start_sc(k, cnt)
            kn = (k + 2) % NRING

            @pl.when(g >= 1)
            def _():
                wait_sc(kn)

            @pl.when(g + 2 < NBLK)
            def _():
                start_in(g + 2, kn)
        return carry

    lax.fori_loop(0, NBLK // NRING, triple, 0)
    for g, k in ((NBLK - 2, 0), (NBLK - 1, 1)):
        wait_in(k)
        cnt = compute(k)
        start_sc(k, cnt)
        wait_sc((k + 2) % NRING)
    wait_sc(1)
    plsc.subcore_barrier()
    pltpu.sync_copy(acc_sh.at[pl.ds(sid * SPS, SPS)],
                    out_hbm.at[c, pl.ds(sid * SPS, SPS)])

    @pl.when(sid == 0)
    def _():
        pltpu.sync_copy(acc_sh.at[pl.ds(NS * SPS, SREM)],
                        out_hbm.at[c, pl.ds(NS * SPS, SREM)])


_sc_scatter = functools.partial(
    pl.kernel,
    mesh=plsc.VectorSubcoreMesh(core_axis_name="c", subcore_axis_name="s"),
    out_type=jax.ShapeDtypeStruct((NC, S, D), jnp.float32),
    scratch_types=[
        pltpu.VMEM((NRING, RB, D), jnp.float32),
        pltpu.VMEM((NRING, CAP, D), jnp.float32),
        pltpu.VMEM((NRING, RB), jnp.float32),
        pltpu.VMEM((NRING, RB), jnp.int32),
        pltpu.VMEM((NRING, 16), jnp.int32),
        pltpu.VMEM((16,), jnp.float32),
        pltpu.SMEM((NRING,), jnp.int32),
        pltpu.VMEM_SHARED((S + 8, D), jnp.float32),
        pltpu.SemaphoreType.DMA,
        pltpu.SemaphoreType.DMA,
        pltpu.SemaphoreType.DMA,
        pltpu.SemaphoreType.DMA,
        pltpu.SemaphoreType.DMA,
        pltpu.SemaphoreType.DMA,
    ],
)(_sc_body)


BS = 1000


def _combine_body(p_ref, o_ref):
    o_ref[...] = p_ref[0] + p_ref[1]


def _combine(partials):
    return pl.pallas_call(
        _combine_body,
        grid=(S // BS,),
        in_specs=[pl.BlockSpec((NC, BS, D), lambda i: (0, i, 0))],
        out_specs=pl.BlockSpec((BS, D), lambda i: (i, 0)),
        out_shape=jax.ShapeDtypeStruct((S, D), jnp.float32),
    )(partials)


def kernel(x, batch_index, W):
    s2d, m, z = _compute_scores(x, W)
    s = s2d.reshape(N)
    mz = jnp.pad(jnp.concatenate([m.reshape(1), z.reshape(1)]), (0, 14))
    zeros = jnp.zeros((S, D), jnp.float32)
    partials = _sc_scatter(x, s, batch_index, mz, zeros)
    return _combine(partials)


# speedup vs baseline: 2.9149x; 2.9149x over previous
import functools

import jax
import jax.numpy as jnp
from jax import lax
from jax.experimental import pallas as pl
from jax.experimental.pallas import tpu as pltpu
from jax.experimental.pallas import tpu_sc as plsc

N = 320000
D = 128
S = 10000


BN = 16000
NB = N // BN


def _scores_body(x_ref, w_ref, s_ref, m_ref, z_ref):
    i = pl.program_id(0)
    xb = x_ref[...]
    wr = w_ref[...]
    sb = lax.dot_general(wr, xb, (((1,), (1,)), ((), ())),
                         preferred_element_type=jnp.float32)
    s_ref[0] = sb
    bm = jnp.max(sb)

    @pl.when(i == 0)
    def _():
        m_ref[0, 0] = bm
        z_ref[0, 0] = jnp.sum(jnp.exp(sb - bm))

    @pl.when(i > 0)
    def _():
        m_old = m_ref[0, 0]
        m_new = jnp.maximum(m_old, bm)
        z_ref[0, 0] = (z_ref[0, 0] * jnp.exp(m_old - m_new)
                       + jnp.sum(jnp.exp(sb - m_new)))
        m_ref[0, 0] = m_new


def _compute_scores(x, w):
    return pl.pallas_call(
        _scores_body,
        grid=(NB,),
        in_specs=[
            pl.BlockSpec((BN, D), lambda i: (i, 0)),
            pl.BlockSpec((1, D), lambda i: (0, 0)),
        ],
        out_specs=[
            pl.BlockSpec((1, 1, BN), lambda i: (i, 0, 0)),
            pl.BlockSpec((1, 1), lambda i: (0, 0), memory_space=pltpu.SMEM),
            pl.BlockSpec((1, 1), lambda i: (0, 0), memory_space=pltpu.SMEM),
        ],
        out_shape=[
            jax.ShapeDtypeStruct((NB, 1, BN), jnp.float32),
            jax.ShapeDtypeStruct((1, 1), jnp.float32),
            jax.ShapeDtypeStruct((1, 1), jnp.float32),
        ],
    )(x, w)


NC = 2
NS = 16
TPT = N // (NC * NS)
RB = 80
NBLK = TPT // RB
SPS = 624
SREM = S - NS * SPS


NRING = 4


def _sc_body(x_hbm, si_hbm, mz_hbm, zeros_hbm, out_hbm,
             xb, sib, ib, mzb, acc_sh, sin0, sin1, sin2, sin3,
             ssc0, ssc1, ssc2, ssc3):
    c = lax.axis_index("c")
    sid = lax.axis_index("s")
    wid = c * NS + sid
    base = wid * TPT

    pltpu.sync_copy(zeros_hbm.at[pl.ds(sid * SPS, SPS)],
                    acc_sh.at[pl.ds(sid * SPS, SPS)])

    @pl.when(sid == 0)
    def _():
        pltpu.sync_copy(zeros_hbm.at[pl.ds(NS * SPS, SREM)],
                        acc_sh.at[pl.ds(NS * SPS, SREM)])

    pltpu.sync_copy(mz_hbm, mzb)
    plsc.subcore_barrier()

    mv = mzb[...]
    m = mv[0]
    z = mv[1]
    sems_in = (sin0, sin1, sin2, sin3)
    sems_sc = (ssc0, ssc1, ssc2, ssc3)

    def start_in(g, k):
        pltpu.async_copy(x_hbm.at[pl.ds(base + g * RB, RB)], xb.at[k],
                         sems_in[k])
        pltpu.async_copy(si_hbm.at[wid * NBLK + g], sib.at[k], sems_in[k])

    def wait_in(k):
        pltpu.make_async_copy(x_hbm.at[pl.ds(0, RB)], xb.at[k],
                              sems_in[k]).wait()
        pltpu.make_async_copy(si_hbm.at[0], sib.at[k], sems_in[k]).wait()

    def start_sc(k):
        pltpu.async_copy(xb.at[k], acc_sh.at[ib.at[k]], sems_sc[k], add=True)

    def wait_sc(k):
        pltpu.make_async_copy(xb.at[k], acc_sh.at[ib.at[k]],
                              sems_sc[k]).wait()

    def compute(k):
        def grp(gg, carry2):
            ib[k, pl.ds(gg * 16, 16)] = sib[k, 0, pl.ds(gg * 16, 16)]
            sv = lax.bitcast_convert_type(sib[k, 1, pl.ds(gg * 16, 16)],
                                          jnp.float32)
            wv = jnp.exp(sv - m) / z
            for r in range(16):
                row = gg * 16 + r
                wr = wv[r]
                for j in range(D // 16):
                    xb[k, row, pl.ds(j * 16, 16)] = (
                        xb[k, row, pl.ds(j * 16, 16)] * wr)
            return carry2

        lax.fori_loop(0, RB // 16, grp, 0)

    start_in(0, 0)
    start_in(1, 1)

    def quad(p, carry):
        for k in range(NRING):
            g = 4 * p + k
            wait_in(k)
            compute(k)
            start_sc(k)
            kn = (k + 2) % NRING

            @pl.when(g >= 2)
            def _():
                wait_sc(kn)

            @pl.when(g + 2 < NBLK)
            def _():
                start_in(g + 2, kn)
        return carry

    lax.fori_loop(0, NBLK // NRING, quad, 0)
    wait_in(0)
    compute(0)
    start_sc(0)
    wait_sc(2)
    wait_sc(3)
    wait_sc(0)
    plsc.subcore_barrier()
    pltpu.sync_copy(acc_sh.at[pl.ds(sid * SPS, SPS)],
                    out_hbm.at[c, pl.ds(sid * SPS, SPS)])

    @pl.when(sid == 0)
    def _():
        pltpu.sync_copy(acc_sh.at[pl.ds(NS * SPS, SREM)],
                        out_hbm.at[c, pl.ds(NS * SPS, SREM)])


_sc_scatter = functools.partial(
    pl.kernel,
    mesh=plsc.VectorSubcoreMesh(core_axis_name="c", subcore_axis_name="s"),
    out_type=jax.ShapeDtypeStruct((NC, S, D), jnp.float32),
    scratch_types=[
        pltpu.VMEM((NRING, RB, D), jnp.float32),
        pltpu.VMEM((NRING, 2, RB), jnp.int32),
        pltpu.VMEM((NRING, RB), jnp.int32),
        pltpu.VMEM((16,), jnp.float32),
        pltpu.VMEM_SHARED((S, D), jnp.float32),
        pltpu.SemaphoreType.DMA,
        pltpu.SemaphoreType.DMA,
        pltpu.SemaphoreType.DMA,
        pltpu.SemaphoreType.DMA,
        pltpu.SemaphoreType.DMA,
        pltpu.SemaphoreType.DMA,
        pltpu.SemaphoreType.DMA,
        pltpu.SemaphoreType.DMA,
    ],
)(_sc_body)


BS = 2000


def _combine_body(p_ref, o_ref):
    o_ref[...] = p_ref[0] + p_ref[1]


def _combine(partials):
    return pl.pallas_call(
        _combine_body,
        grid=(S // BS,),
        in_specs=[pl.BlockSpec((NC, BS, D), lambda i: (0, i, 0))],
        out_specs=pl.BlockSpec((BS, D), lambda i: (i, 0)),
        out_shape=jax.ShapeDtypeStruct((S, D), jnp.float32),
    )(partials)


def kernel(x, batch_index, W):
    s2d, m, z = _compute_scores(x, W)
    sbits = lax.bitcast_convert_type(s2d.reshape(N), jnp.int32)
    si = jnp.stack(
        [batch_index.reshape(N // RB, RB), sbits.reshape(N // RB, RB)],
        axis=1)
    mz = jnp.pad(jnp.concatenate([m.reshape(1), z.reshape(1)]), (0, 14))
    zeros = jnp.zeros((S, D), jnp.float32)
    partials = _sc_scatter(x, si, mz, zeros)
    return _combine(partials)


# speedup vs baseline: 2.9923x; 1.0265x over previous
import functools

import jax
import jax.numpy as jnp
from jax import lax
from jax.experimental import pallas as pl
from jax.experimental.pallas import tpu as pltpu
from jax.experimental.pallas import tpu_sc as plsc

N = 320000
D = 128
S = 10000


BN = 16000
NB = N // BN


def _scores_body(x_ref, w_ref, s_ref, m_ref, z_ref):
    i = pl.program_id(0)
    xb = x_ref[...]
    wr = w_ref[...]
    sb = lax.dot_general(wr, xb, (((1,), (1,)), ((), ())),
                         preferred_element_type=jnp.float32)
    s_ref[0] = sb
    bm = jnp.max(sb)

    @pl.when(i == 0)
    def _():
        m_ref[0, 0] = bm
        z_ref[0, 0] = jnp.sum(jnp.exp(sb - bm))

    @pl.when(i > 0)
    def _():
        m_old = m_ref[0, 0]
        m_new = jnp.maximum(m_old, bm)
        z_ref[0, 0] = (z_ref[0, 0] * jnp.exp(m_old - m_new)
                       + jnp.sum(jnp.exp(sb - m_new)))
        m_ref[0, 0] = m_new


def _compute_scores(x, w):
    return pl.pallas_call(
        _scores_body,
        grid=(NB,),
        in_specs=[
            pl.BlockSpec((BN, D), lambda i: (i, 0)),
            pl.BlockSpec((1, D), lambda i: (0, 0)),
        ],
        out_specs=[
            pl.BlockSpec((1, 1, BN), lambda i: (i, 0, 0)),
            pl.BlockSpec((1, 1), lambda i: (0, 0), memory_space=pltpu.SMEM),
            pl.BlockSpec((1, 1), lambda i: (0, 0), memory_space=pltpu.SMEM),
        ],
        out_shape=[
            jax.ShapeDtypeStruct((NB, 1, BN), jnp.float32),
            jax.ShapeDtypeStruct((1, 1), jnp.float32),
            jax.ShapeDtypeStruct((1, 1), jnp.float32),
        ],
    )(x, w)


NC = 2
NS = 16
TPT = N // (NC * NS)
RB = 80
NBLK = TPT // RB
SPS = 624
SREM = S - NS * SPS


NRING = 4


def _sc_body(x_hbm, s_hbm, idx_hbm, mz_hbm, zeros_hbm, out_hbm,
             xb, sb, ib, mzb, acc_sh, sin0, sin1, sin2, sin3,
             ssc0, ssc1, ssc2, ssc3):
    c = lax.axis_index("c")
    sid = lax.axis_index("s")
    wid = c * NS + sid
    base = wid * TPT

    pltpu.sync_copy(zeros_hbm.at[pl.ds(sid * SPS, SPS)],
                    acc_sh.at[pl.ds(sid * SPS, SPS)])

    @pl.when(sid == 0)
    def _():
        pltpu.sync_copy(zeros_hbm.at[pl.ds(NS * SPS, SREM)],
                        acc_sh.at[pl.ds(NS * SPS, SREM)])

    pltpu.sync_copy(mz_hbm, mzb)
    plsc.subcore_barrier()

    mv = mzb[...]
    m = mv[0]
    z = mv[1]
    sems_in = (sin0, sin1, sin2, sin3)
    sems_sc = (ssc0, ssc1, ssc2, ssc3)

    def start_in(g, k):
        b0 = base + g * RB
        pltpu.async_copy(x_hbm.at[pl.ds(b0, RB)], xb.at[k], sems_in[k])
        pltpu.async_copy(s_hbm.at[pl.ds(b0, RB)], sb.at[k], sems_in[k])
        pltpu.async_copy(idx_hbm.at[pl.ds(b0, RB)], ib.at[k], sems_in[k])

    def wait_in(k):
        pltpu.make_async_copy(x_hbm.at[pl.ds(0, RB)], xb.at[k],
                              sems_in[k]).wait()
        pltpu.make_async_copy(s_hbm.at[pl.ds(0, RB)], sb.at[k],
                              sems_in[k]).wait()
        pltpu.make_async_copy(idx_hbm.at[pl.ds(0, RB)], ib.at[k],
                              sems_in[k]).wait()

    def start_sc(k):
        pltpu.async_copy(xb.at[k], acc_sh.at[ib.at[k]], sems_sc[k], add=True)

    def wait_sc(k):
        pltpu.make_async_copy(xb.at[k], acc_sh.at[ib.at[k]],
                              sems_sc[k]).wait()

    def compute(k):
        def grp(gg, carry2):
            sv = sb[k, pl.ds(gg * 16, 16)]
            wv = jnp.exp(sv - m) / z
            for r in range(16):
                row = gg * 16 + r
                wr = wv[r]
                for j in range(D // 16):
                    xb[k, row, pl.ds(j * 16, 16)] = (
                        xb[k, row, pl.ds(j * 16, 16)] * wr)
            return carry2

        lax.fori_loop(0, RB // 16, grp, 0)

    start_in(0, 0)
    start_in(1, 1)

    def quad(p, carry):
        for k in range(NRING):
            g = 4 * p + k
            wait_in(k)
            compute(k)
            start_sc(k)
            kn = (k + 2) % NRING

            @pl.when(g >= 2)
            def _():
                wait_sc(kn)

            @pl.when(g + 2 < NBLK)
            def _():
                start_in(g + 2, kn)
        return carry

    lax.fori_loop(0, NBLK // NRING, quad, 0)
    wait_in(0)
    compute(0)
    start_sc(0)
    wait_sc(2)
    wait_sc(3)
    wait_sc(0)
    plsc.subcore_barrier()
    pltpu.sync_copy(acc_sh.at[pl.ds(sid * SPS, SPS)],
                    out_hbm.at[c, pl.ds(sid * SPS, SPS)])

    @pl.when(sid == 0)
    def _():
        pltpu.sync_copy(acc_sh.at[pl.ds(NS * SPS, SREM)],
                        out_hbm.at[c, pl.ds(NS * SPS, SREM)])


_sc_scatter = functools.partial(
    pl.kernel,
    mesh=plsc.VectorSubcoreMesh(core_axis_name="c", subcore_axis_name="s"),
    out_type=jax.ShapeDtypeStruct((NC, S, D), jnp.float32),
    scratch_types=[
        pltpu.VMEM((NRING, RB, D), jnp.float32),
        pltpu.VMEM((NRING, RB), jnp.float32),
        pltpu.VMEM((NRING, RB), jnp.int32),
        pltpu.VMEM((16,), jnp.float32),
        pltpu.VMEM_SHARED((S, D), jnp.float32),
        pltpu.SemaphoreType.DMA,
        pltpu.SemaphoreType.DMA,
        pltpu.SemaphoreType.DMA,
        pltpu.SemaphoreType.DMA,
        pltpu.SemaphoreType.DMA,
        pltpu.SemaphoreType.DMA,
        pltpu.SemaphoreType.DMA,
        pltpu.SemaphoreType.DMA,
    ],
)(_sc_body)


BS = 2000


def _combine_body(p_ref, o_ref):
    o_ref[...] = p_ref[0] + p_ref[1]


def _combine(partials):
    return pl.pallas_call(
        _combine_body,
        grid=(S // BS,),
        in_specs=[pl.BlockSpec((NC, BS, D), lambda i: (0, i, 0))],
        out_specs=pl.BlockSpec((BS, D), lambda i: (i, 0)),
        out_shape=jax.ShapeDtypeStruct((S, D), jnp.float32),
    )(partials)


def kernel(x, batch_index, W):
    s2d, m, z = _compute_scores(x, W)
    s = s2d.reshape(N)
    mz = jnp.pad(jnp.concatenate([m.reshape(1), z.reshape(1)]), (0, 14))
    zeros = jnp.zeros((S, D), jnp.float32)
    partials = _sc_scatter(x, s, batch_index, mz, zeros)
    return _combine(partials)


# speedup vs baseline: 2.9990x; 1.0023x over previous
import functools

import jax
import jax.numpy as jnp
from jax import lax
from jax.experimental import pallas as pl
from jax.experimental.pallas import tpu as pltpu
from jax.experimental.pallas import tpu_sc as plsc

N = 320000
D = 128
S = 10000


BN = 32000
NB = N // BN


def _scores_body(x_ref, w_ref, s_ref, m_ref, z_ref):
    i = pl.program_id(0)
    xb = x_ref[...]
    wr = w_ref[...]
    sb = lax.dot_general(wr, xb, (((1,), (1,)), ((), ())),
                         preferred_element_type=jnp.float32)
    s_ref[0] = sb
    bm = jnp.max(sb)

    @pl.when(i == 0)
    def _():
        m_ref[0, 0] = bm
        z_ref[0, 0] = jnp.sum(jnp.exp(sb - bm))

    @pl.when(i > 0)
    def _():
        m_old = m_ref[0, 0]
        m_new = jnp.maximum(m_old, bm)
        z_ref[0, 0] = (z_ref[0, 0] * jnp.exp(m_old - m_new)
                       + jnp.sum(jnp.exp(sb - m_new)))
        m_ref[0, 0] = m_new


def _compute_scores(x, w):
    return pl.pallas_call(
        _scores_body,
        grid=(NB,),
        in_specs=[
            pl.BlockSpec((BN, D), lambda i: (i, 0)),
            pl.BlockSpec((1, D), lambda i: (0, 0)),
        ],
        out_specs=[
            pl.BlockSpec((1, 1, BN), lambda i: (i, 0, 0)),
            pl.BlockSpec((1, 1), lambda i: (0, 0), memory_space=pltpu.SMEM),
            pl.BlockSpec((1, 1), lambda i: (0, 0), memory_space=pltpu.SMEM),
        ],
        out_shape=[
            jax.ShapeDtypeStruct((NB, 1, BN), jnp.float32),
            jax.ShapeDtypeStruct((1, 1), jnp.float32),
            jax.ShapeDtypeStruct((1, 1), jnp.float32),
        ],
    )(x, w)


NC = 2
NS = 16
TPT = N // (NC * NS)
RB = 80
NBLK = TPT // RB
SPS = 624
SREM = S - NS * SPS


NRING = 4


def _sc_body(x_hbm, s_hbm, idx_hbm, mz_hbm, zeros_hbm, out_hbm,
             xb, sb, ib, mzb, acc_sh, sin0, sin1, sin2, sin3,
             ssc0, ssc1, ssc2, ssc3):
    c = lax.axis_index("c")
    sid = lax.axis_index("s")
    wid = c * NS + sid
    base = wid * TPT

    pltpu.sync_copy(zeros_hbm.at[pl.ds(sid * SPS, SPS)],
                    acc_sh.at[pl.ds(sid * SPS, SPS)])

    @pl.when(sid == 0)
    def _():
        pltpu.sync_copy(zeros_hbm.at[pl.ds(NS * SPS, SREM)],
                        acc_sh.at[pl.ds(NS * SPS, SREM)])

    pltpu.sync_copy(mz_hbm, mzb)
    plsc.subcore_barrier()

    mv = mzb[...]
    m = mv[0]
    z = mv[1]
    sems_in = (sin0, sin1, sin2, sin3)
    sems_sc = (ssc0, ssc1, ssc2, ssc3)

    def start_in(g, k):
        b0 = base + g * RB
        pltpu.async_copy(x_hbm.at[pl.ds(b0, RB)], xb.at[k], sems_in[k])
        pltpu.async_copy(s_hbm.at[pl.ds(b0, RB)], sb.at[k], sems_in[k])
        pltpu.async_copy(idx_hbm.at[pl.ds(b0, RB)], ib.at[k], sems_in[k])

    def wait_in(k):
        pltpu.make_async_copy(x_hbm.at[pl.ds(0, RB)], xb.at[k],
                              sems_in[k]).wait()
        pltpu.make_async_copy(s_hbm.at[pl.ds(0, RB)], sb.at[k],
                              sems_in[k]).wait()
        pltpu.make_async_copy(idx_hbm.at[pl.ds(0, RB)], ib.at[k],
                              sems_in[k]).wait()

    def start_sc(k):
        pltpu.async_copy(xb.at[k], acc_sh.at[ib.at[k]], sems_sc[k], add=True)

    def wait_sc(k):
        pltpu.make_async_copy(xb.at[k], acc_sh.at[ib.at[k]],
                              sems_sc[k]).wait()

    def compute(k):
        def grp(gg, carry2):
            sv = sb[k, pl.ds(gg * 16, 16)]
            wv = jnp.exp(sv - m) / z
            for r in range(16):
                row = gg * 16 + r
                wr = wv[r]
                for j in range(D // 16):
                    xb[k, row, pl.ds(j * 16, 16)] = (
                        xb[k, row, pl.ds(j * 16, 16)] * wr)
            return carry2

        lax.fori_loop(0, RB // 16, grp, 0)

    start_in(0, 0)
    start_in(1, 1)

    def quad(p, carry):
        for k in range(NRING):
            g = 4 * p + k
            wait_in(k)
            compute(k)
            start_sc(k)
            kn = (k + 2) % NRING

            @pl.when(g >= 2)
            def _():
                wait_sc(kn)

            @pl.when(g + 2 < NBLK)
            def _():
                start_in(g + 2, kn)
        return carry

    lax.fori_loop(0, NBLK // NRING, quad, 0)
    wait_in(0)
    compute(0)
    start_sc(0)
    wait_sc(2)
    wait_sc(3)
    wait_sc(0)
    plsc.subcore_barrier()
    pltpu.sync_copy(acc_sh.at[pl.ds(sid * SPS, SPS)],
                    out_hbm.at[c, pl.ds(sid * SPS, SPS)])

    @pl.when(sid == 0)
    def _():
        pltpu.sync_copy(acc_sh.at[pl.ds(NS * SPS, SREM)],
                        out_hbm.at[c, pl.ds(NS * SPS, SREM)])


_sc_scatter = functools.partial(
    pl.kernel,
    mesh=plsc.VectorSubcoreMesh(core_axis_name="c", subcore_axis_name="s"),
    out_type=jax.ShapeDtypeStruct((NC, S, D), jnp.float32),
    scratch_types=[
        pltpu.VMEM((NRING, RB, D), jnp.float32),
        pltpu.VMEM((NRING, RB), jnp.float32),
        pltpu.VMEM((NRING, RB), jnp.int32),
        pltpu.VMEM((16,), jnp.float32),
        pltpu.VMEM_SHARED((S, D), jnp.float32),
        pltpu.SemaphoreType.DMA,
        pltpu.SemaphoreType.DMA,
        pltpu.SemaphoreType.DMA,
        pltpu.SemaphoreType.DMA,
        pltpu.SemaphoreType.DMA,
        pltpu.SemaphoreType.DMA,
        pltpu.SemaphoreType.DMA,
        pltpu.SemaphoreType.DMA,
    ],
)(_sc_body)


BS = 2000


def _combine_body(p_ref, o_ref):
    o_ref[...] = p_ref[0] + p_ref[1]


def _combine(partials):
    return pl.pallas_call(
        _combine_body,
        grid=(S // BS,),
        in_specs=[pl.BlockSpec((NC, BS, D), lambda i: (0, i, 0))],
        out_specs=pl.BlockSpec((BS, D), lambda i: (i, 0)),
        out_shape=jax.ShapeDtypeStruct((S, D), jnp.float32),
    )(partials)


def kernel(x, batch_index, W):
    s2d, m, z = _compute_scores(x, W)
    s = s2d.reshape(N)
    mz = jnp.pad(jnp.concatenate([m.reshape(1), z.reshape(1)]), (0, 14))
    zeros = jnp.zeros((S, D), jnp.float32)
    partials = _sc_scatter(x, s, batch_index, mz, zeros)
    return _combine(partials)
